# baseline (speedup 1.0000x reference)
import functools

import jax
import jax.numpy as jnp
from jax import lax
from jax.experimental import pallas as pl
from jax.experimental.pallas import tpu as pltpu
from jax.experimental.pallas import tpu_sc as plsc

EMBED_DIM = 64
NC = 2
NS = 16
NW = NC * NS
G = 128


@functools.partial(jax.jit, static_argnums=(2, 3))
def _emb(idx, weight, b_per_w, ng):
    mesh = plsc.VectorSubcoreMesh(
        core_axis_name="c", subcore_axis_name="s", num_cores=NC,
        num_subcores=NS)

    @functools.partial(
        pl.kernel,
        out_type=jax.ShapeDtypeStruct((NW * b_per_w, EMBED_DIM), jnp.float32),
        mesh=mesh,
        scratch_types=[
            pltpu.VMEM((ng, G), jnp.int32),
            pltpu.VMEM((G, EMBED_DIM), jnp.float32),
            pltpu.SemaphoreType.DMA,
        ],
        compiler_params=pltpu.CompilerParams(use_tc_tiling_on_sc=False),
    )
    def body(idx_hbm, w_hbm, out_hbm, idx_v, rows_v, sem):
        wid = lax.axis_index("s") * NC + lax.axis_index("c")
        base = wid * b_per_w
        pltpu.sync_copy(idx_hbm.at[wid], idx_v)

        def step(g, carry):
            pltpu.async_copy(w_hbm.at[idx_v.at[g]], rows_v, sem).wait()
            pltpu.sync_copy(rows_v, out_hbm.at[pl.ds(base + g * G, G)])
            return carry

        lax.fori_loop(0, ng, step, 0)

    return body(idx, weight)


def kernel(x, weight):
    b, s = x.shape
    total = b * s
    b_per_w = total // NW
    ng = b_per_w // G
    idx = x.reshape(NW, ng, G).astype(jnp.int32)
    out = _emb(idx, weight, b_per_w, ng)
    return out.reshape(b, s, EMBED_DIM)


# speedup vs baseline: 1.1140x; 1.1140x over previous
import functools

import jax
import jax.numpy as jnp
from jax import lax
from jax.experimental import pallas as pl
from jax.experimental.pallas import tpu as pltpu
from jax.experimental.pallas import tpu_sc as plsc

EMBED_DIM = 64
NC = 2
NS = 16
NW = NC * NS
G = 128
K = 4


@functools.partial(jax.jit, static_argnums=(2, 3))
def _emb(idx, weight, b_per_w, ng):
    nchunks = ng // K
    npairs = nchunks // 2
    mesh = plsc.VectorSubcoreMesh(
        core_axis_name="c", subcore_axis_name="s", num_cores=NC,
        num_subcores=NS)

    @functools.partial(
        pl.kernel,
        out_type=jax.ShapeDtypeStruct((NW * b_per_w, EMBED_DIM), jnp.float32),
        mesh=mesh,
        scratch_types=[
            pltpu.VMEM((ng, G), jnp.int32),
            pltpu.VMEM((K, G, EMBED_DIM), jnp.float32),
            pltpu.VMEM((K, G, EMBED_DIM), jnp.float32),
            pltpu.SemaphoreType.DMA,
            pltpu.SemaphoreType.DMA,
            pltpu.SemaphoreType.DMA,
            pltpu.SemaphoreType.DMA,
        ],
        compiler_params=pltpu.CompilerParams(use_tc_tiling_on_sc=False),
    )
    def body(idx_hbm, w_hbm, out_hbm, idx_v, buf_a, buf_b, sem_ga, sem_gb,
             sem_wa, sem_wb):
        wid = lax.axis_index("s") * NC + lax.axis_index("c")
        base = wid * b_per_w
        pltpu.sync_copy(idx_hbm.at[wid], idx_v)

        def fire_gathers(c, buf, sem):
            for j in range(K):
                pltpu.async_copy(w_hbm.at[idx_v.at[c * K + j]], buf.at[j], sem)

        def drain_gathers(c, buf, sem):
            for j in range(K):
                pltpu.make_async_copy(
                    w_hbm.at[idx_v.at[c * K + j]], buf.at[j], sem).wait()

        def fire_wbs(c, buf, sem):
            for j in range(K):
                pltpu.async_copy(
                    buf.at[j],
                    out_hbm.at[pl.ds(base + (c * K + j) * G, G)], sem)

        def drain_wbs(c, buf, sem):
            for j in range(K):
                pltpu.make_async_copy(
                    buf.at[j],
                    out_hbm.at[pl.ds(base + (c * K + j) * G, G)], sem).wait()

        def half(c0, c1, buf0, buf1, sem_g1, sem_g0_wait, sem_w0):
            fire_gathers(c1, buf1, sem_g1)
            drain_gathers(c0, buf0, sem_g0_wait)
            fire_wbs(c0, buf0, sem_w0)
            drain_wbs(c0, buf0, sem_w0)

        fire_gathers(0, buf_a, sem_ga)

        def pair(p, carry):
            c0 = 2 * p
            half(c0, c0 + 1, buf_a, buf_b, sem_gb, sem_ga, sem_wa)
            half(c0 + 1, c0 + 2, buf_b, buf_a, sem_ga, sem_gb, sem_wb)
            return carry

        lax.fori_loop(0, npairs - 1, pair, 0)

        c0 = 2 * (npairs - 1)
        half(c0, c0 + 1, buf_a, buf_b, sem_gb, sem_ga, sem_wa)
        drain_gathers(c0 + 1, buf_b, sem_gb)
        fire_wbs(c0 + 1, buf_b, sem_wb)
        drain_wbs(c0 + 1, buf_b, sem_wb)

    return body(idx, weight)


def kernel(x, weight):
    b, s = x.shape
    total = b * s
    b_per_w = total // NW
    ng = b_per_w // G
    idx = x.reshape(NW, ng, G).astype(jnp.int32)
    out = _emb(idx, weight, b_per_w, ng)
    return out.reshape(b, s, EMBED_DIM)
